# Initial kernel scaffold; baseline (speedup 1.0000x reference)
import jax
import jax.numpy as jnp
from jax.experimental import pallas as pl


def kernel(inputs, categ_bias, output_layer_bias, moving_mean, moving_norm):
    raise NotImplementedError("write your pallas kernel here")


# speedup vs baseline: 1.0200x; 1.0200x over previous
import jax
import jax.numpy as jnp
from jax import lax
from jax.experimental import pallas as pl
from jax.experimental.pallas import tpu as pltpu
from jax.experimental.pallas import tpu_sc as plsc

NC = 2
NS = 16
NW = NC * NS
B = 16384
PER_W = B // NW
CHUNK = 128
NCHUNK = PER_W // CHUNK
L = 16


def _gather_bias_body(table_hbm, idx_hbm, bias_hbm, out_hbm,
                      idx_v, rows_v, bias_v, sem):
    wid = lax.axis_index("s") * NC + lax.axis_index("c")
    pltpu.sync_copy(idx_hbm.at[wid], idx_v)
    pltpu.sync_copy(bias_hbm, bias_v)
    copies = [
        pltpu.async_copy(table_hbm.at[idx_v.at[j]], rows_v.at[j], sem)
        for j in range(NCHUNK)
    ]
    for c in copies:
        c.wait()
    bv = bias_v[...]
    for j in range(NCHUNK):
        for i in range(CHUNK // L):
            sl = pl.ds(i * L, L)
            rows_v[j, sl] = rows_v[j, sl] + bv
    pltpu.sync_copy(rows_v, out_hbm.at[wid])


def kernel(inputs, categ_bias, output_layer_bias, moving_mean, moving_norm):
    idx = inputs[:, 0].astype(jnp.int32).reshape(NW, NCHUNK, CHUNK)
    table = categ_bias.reshape(-1)
    bias16 = jnp.broadcast_to(output_layer_bias.reshape(1), (L,))
    run = pl.kernel(
        _gather_bias_body,
        out_type=jax.ShapeDtypeStruct((NW, NCHUNK, CHUNK), jnp.float32),
        mesh=plsc.VectorSubcoreMesh(core_axis_name="c", subcore_axis_name="s"),
        scratch_types=[
            pltpu.VMEM((NCHUNK, CHUNK), jnp.int32),
            pltpu.VMEM((NCHUNK, CHUNK), jnp.float32),
            pltpu.VMEM((L,), jnp.float32),
            pltpu.SemaphoreType.DMA,
        ],
    )
    out = run(table, idx, bias16)
    return out.reshape(B, 1)


# speedup vs baseline: 2.4834x; 2.4346x over previous
import jax
import jax.numpy as jnp
from jax import lax
from jax.experimental import pallas as pl
from jax.experimental.pallas import tpu as pltpu
from jax.experimental.pallas import tpu_sc as plsc

NC = 2
NS = 16
NW = NC * NS
B = 16384
PER_W = B // NW
CHUNK = 128
NCHUNK = PER_W // CHUNK
L = 16
V = 1000000
SLAB = 62528
LAST = V - 15 * SLAB


def _gather_body(table_hbm, idx_hbm, bias_hbm, out_hbm,
                 spt, idx_v, rows_v, bias_v, sem):
    cid = lax.axis_index("c")
    sid = lax.axis_index("s")
    wid = sid * NC + cid
    @pl.when(sid == 0)
    def _():
        pltpu.sync_copy(table_hbm.at[0], spt)

    plsc.subcore_barrier()
    pltpu.sync_copy(idx_hbm.at[wid], idx_v)
    pltpu.sync_copy(bias_hbm, bias_v)
    copies = [
        pltpu.async_copy(spt.at[idx_v.at[j]], rows_v.at[j], sem)
        for j in range(NCHUNK)
    ]
    for c in copies:
        c.wait()
    bv = bias_v[...]
    for j in range(NCHUNK):
        for i in range(CHUNK // L):
            sl = pl.ds(i * L, L)
            rows_v[j, sl] = rows_v[j, sl] + bv
    pltpu.sync_copy(rows_v, out_hbm.at[wid])


def kernel(inputs, categ_bias, output_layer_bias, moving_mean, moving_norm):
    idx = inputs[:, 0].astype(jnp.int32).reshape(NW, NCHUNK, CHUNK)
    table = jnp.swapaxes(categ_bias, 0, 1)
    bias16 = jnp.broadcast_to(output_layer_bias.reshape(1), (L,))
    run = pl.kernel(
        _gather_body,
        out_type=jax.ShapeDtypeStruct((NW, NCHUNK, CHUNK), jnp.float32),
        mesh=plsc.VectorSubcoreMesh(core_axis_name="c", subcore_axis_name="s"),
        scratch_types=[
            pltpu.VMEM_SHARED((V,), jnp.float32),
            pltpu.VMEM((NCHUNK, CHUNK), jnp.int32),
            pltpu.VMEM((NCHUNK, CHUNK), jnp.float32),
            pltpu.VMEM((L,), jnp.float32),
            pltpu.SemaphoreType.DMA,
        ],
    )
    out = run(table, idx, bias16)
    return out.reshape(B, 1)


# speedup vs baseline: 2.9956x; 1.2062x over previous
import jax
import jax.numpy as jnp
from jax import lax
from jax.experimental import pallas as pl
from jax.experimental.pallas import tpu as pltpu
from jax.experimental.pallas import tpu_sc as plsc

NC = 2
NS = 16
NW = NC * NS
B = 16384
PER_W = B // NW
CHUNK = 128
NCHUNK = PER_W // CHUNK
L = 16
V = 1000000
SLAB = 62528
LAST = V - 15 * SLAB


def _gather_body(table_hbm, idx_hbm, bias_hbm, out_hbm,
                 idx_v, rows_v, bias_v, sem):
    cid = lax.axis_index("c")
    sid = lax.axis_index("s")
    wid = sid * NC + cid
    tab1d = table_hbm.at[0]
    pltpu.sync_copy(idx_hbm.at[wid], idx_v)
    pltpu.sync_copy(bias_hbm, bias_v)
    copies = [
        pltpu.async_copy(tab1d.at[idx_v.at[j]], rows_v.at[j], sem)
        for j in range(NCHUNK)
    ]
    for c in copies:
        c.wait()
    bv = bias_v[...]
    for j in range(NCHUNK):
        for i in range(CHUNK // L):
            sl = pl.ds(i * L, L)
            rows_v[j, sl] = rows_v[j, sl] + bv
    pltpu.sync_copy(rows_v, out_hbm.at[wid])


def kernel(inputs, categ_bias, output_layer_bias, moving_mean, moving_norm):
    idx = inputs[:, 0].astype(jnp.int32).reshape(NW, NCHUNK, CHUNK)
    table = jnp.swapaxes(categ_bias, 0, 1)
    bias16 = jnp.broadcast_to(output_layer_bias.reshape(1), (L,))
    run = pl.kernel(
        _gather_body,
        out_type=jax.ShapeDtypeStruct((NW, NCHUNK, CHUNK), jnp.float32),
        mesh=plsc.VectorSubcoreMesh(core_axis_name="c", subcore_axis_name="s"),
        scratch_types=[
            pltpu.VMEM((NCHUNK, CHUNK), jnp.int32),
            pltpu.VMEM((NCHUNK, CHUNK), jnp.float32),
            pltpu.VMEM((L,), jnp.float32),
            pltpu.SemaphoreType.DMA,
        ],
    )
    out = run(table, idx, bias16)
    return out.reshape(B, 1)
